# baseline (speedup 1.0000x reference)
import jax
import jax.numpy as jnp
from jax.experimental import pallas as pl
from jax.experimental.pallas import tpu as pltpu

_VMEM_LIMIT = 60 * 1024 * 1024
_ROW_TILE = 256


def _round_up(x, m):
    return ((x + m - 1) // m) * m


def _pad2d(x, rows, cols):
    r, c = x.shape
    if r == rows and c == cols:
        return x
    return jnp.pad(x, ((0, rows - r), (0, cols - c)))


def _fused_kernel(inp_ref, x1_ref, x2_ref, out_ref, stash_ref, w_ref):
    i = pl.program_id(0)
    nt = pl.num_programs(0) // 2
    rt = inp_ref.shape[0]

    @pl.when(i < nt)
    def _phase1():
        row0 = pl.multiple_of(i * rt, rt)
        a = inp_ref[...].astype(jnp.bfloat16)
        stash_ref[pl.ds(row0, rt), :] = a
        w_ref[pl.ds(row0, rt), :] = jnp.dot(
            a, x2_ref[...], preferred_element_type=jnp.float32
        ).astype(jnp.bfloat16)

    @pl.when(i >= nt)
    def _phase2():
        j = i - nt
        row0 = pl.multiple_of(j * rt, rt)
        out_ref[pl.ds(row0, rt), :] = (
            jnp.dot(
                stash_ref[pl.ds(row0, rt), :],
                w_ref[...],
                preferred_element_type=jnp.float32,
            )
            + jnp.dot(
                x1_ref[...].astype(jnp.bfloat16),
                x2_ref[...],
                preferred_element_type=jnp.float32,
            )
        )


def _forward(inp_p, x1_p, x2b, row_tile):
    Np = inp_p.shape[0]
    Mp = x2b.shape[1]
    nt = Np // row_tile
    grid = (2 * nt,)
    last = nt - 1

    def inp_map(i):
        return (jnp.minimum(i, last), 0)

    def x1_map(i):
        return (jnp.maximum(i - nt, 0), 0)

    return pl.pallas_call(
        _fused_kernel,
        out_shape=jax.ShapeDtypeStruct((Np, Mp), jnp.float32),
        grid=grid,
        in_specs=[
            pl.BlockSpec((row_tile, Np), inp_map),
            pl.BlockSpec((row_tile, Np), x1_map),
            pl.BlockSpec(
                (Np, Mp), lambda i: (0, 0),
                pipeline_mode=pl.Buffered(buffer_count=1),
            ),
        ],
        out_specs=pl.BlockSpec(
            (Np, Mp), lambda i: (0, 0),
            pipeline_mode=pl.Buffered(buffer_count=1),
        ),
        scratch_shapes=[
            pltpu.VMEM((Np, Np), jnp.bfloat16),
            pltpu.VMEM((Np, Mp), jnp.bfloat16),
        ],
        compiler_params=pltpu.CompilerParams(
            dimension_semantics=("arbitrary",),
            vmem_limit_bytes=_VMEM_LIMIT,
        ),
        cost_estimate=pl.CostEstimate(
            flops=6 * Np * Np * Mp,
            transcendentals=0,
            bytes_accessed=4 * (2 * Np * Np + 2 * Np * Mp),
        ),
    )(inp_p, x1_p, x2b)


def kernel(x1, x2, inp):
    N, N2 = inp.shape
    assert N == N2
    M = x2.shape[1]

    Mp = _round_up(max(M, 128), 128)
    Np = _round_up(N, 512)

    inp_p = _pad2d(inp.astype(jnp.float32), Np, Np)
    x1_p = _pad2d(x1.astype(jnp.float32), Np, Np)
    x2b = _pad2d(x2.astype(jnp.float32), Np, Mp).astype(jnp.bfloat16)

    out_p = _forward(inp_p, x1_p, x2b, _ROW_TILE)
    return out_p[:N, :M]


# speedup vs baseline: 1.1104x; 1.1104x over previous
import jax
import jax.numpy as jnp
from jax.experimental import pallas as pl
from jax.experimental.pallas import tpu as pltpu

_VMEM_LIMIT = 60 * 1024 * 1024
_ROW_TILE = 256
_TAIL_TILE = 1024


def _round_up(x, m):
    return ((x + m - 1) // m) * m


def _pad2d(x, rows, cols):
    r, c = x.shape
    if r == rows and c == cols:
        return x
    return jnp.pad(x, ((0, rows - r), (0, cols - c)))


def _make_fused_kernel(nt):
    def _fused_kernel(
        inp_ref, x1_ref, x2_ref, out_ref, stash_ref, w_ref, y_ref, x2b_ref
    ):
        i = pl.program_id(0)
        rt = inp_ref.shape[0]

        @pl.when(i == 0)
        def _cast_x2():
            x2b_ref[...] = x2_ref[...].astype(jnp.bfloat16)

        @pl.when(i < nt)
        def _phase1():
            row0 = pl.multiple_of(i * rt, rt)
            a = inp_ref[...].astype(jnp.bfloat16)
            stash_ref[pl.ds(row0, rt), :] = a
            w_ref[pl.ds(row0, rt), :] = jnp.dot(
                a, x2b_ref[...], preferred_element_type=jnp.float32
            ).astype(jnp.bfloat16)
            y_ref[pl.ds(row0, rt), :] = jnp.dot(
                x1_ref[...].astype(jnp.bfloat16),
                x2b_ref[...],
                preferred_element_type=jnp.float32,
            )

        @pl.when(i >= nt)
        def _tail():
            j = i - nt
            row0 = pl.multiple_of(j * _TAIL_TILE, _TAIL_TILE)
            out_ref[...] = (
                jnp.dot(
                    stash_ref[pl.ds(row0, _TAIL_TILE), :],
                    w_ref[...],
                    preferred_element_type=jnp.float32,
                )
                + y_ref[pl.ds(row0, _TAIL_TILE), :]
            )

    return _fused_kernel


def _forward(inp_p, x1_p, x2_p, row_tile):
    Np = inp_p.shape[0]
    Mp = x2_p.shape[1]
    nt = Np // row_tile
    n_tail = Np // _TAIL_TILE
    grid = (nt + n_tail,)
    last = nt - 1

    def phase1_map(i):
        return (jnp.minimum(i, last), 0)

    def out_map(i):
        return (jnp.maximum(i - nt, 0), 0)

    return pl.pallas_call(
        _make_fused_kernel(nt),
        out_shape=jax.ShapeDtypeStruct((Np, Mp), jnp.float32),
        grid=grid,
        in_specs=[
            pl.BlockSpec((row_tile, Np), phase1_map),
            pl.BlockSpec((row_tile, Np), phase1_map),
            pl.BlockSpec(
                (Np, Mp), lambda i: (0, 0),
                pipeline_mode=pl.Buffered(buffer_count=1),
            ),
        ],
        out_specs=pl.BlockSpec((_TAIL_TILE, Mp), out_map),
        scratch_shapes=[
            pltpu.VMEM((Np, Np), jnp.bfloat16),
            pltpu.VMEM((Np, Mp), jnp.bfloat16),
            pltpu.VMEM((Np, Mp), jnp.float32),
            pltpu.VMEM((Np, Mp), jnp.bfloat16),
        ],
        compiler_params=pltpu.CompilerParams(
            dimension_semantics=("arbitrary",),
            vmem_limit_bytes=_VMEM_LIMIT,
        ),
        cost_estimate=pl.CostEstimate(
            flops=6 * Np * Np * Mp,
            transcendentals=0,
            bytes_accessed=4 * (2 * Np * Np + 2 * Np * Mp),
        ),
    )(inp_p, x1_p, x2_p)


def kernel(x1, x2, inp):
    N, N2 = inp.shape
    assert N == N2
    M = x2.shape[1]

    Mp = _round_up(max(M, 128), 128)
    Np = _round_up(N, 1024)

    inp_p = _pad2d(inp.astype(jnp.float32), Np, Np)
    x1_p = _pad2d(x1.astype(jnp.float32), Np, Np)
    x2_p = _pad2d(x2.astype(jnp.float32), Np, Mp)

    out_p = _forward(inp_p, x1_p, x2_p, _ROW_TILE)
    return out_p[:N, :M]
